# baseline (speedup 1.0000x reference)
import functools

import jax
import jax.numpy as jnp
from jax import lax
from jax.experimental import pallas as pl
from jax.experimental.pallas import tpu as pltpu
from jax.experimental.pallas import tpu_sc as plsc

N = 1_000_000
CA = 4000
_BOUNDS = (0, 280_000, 560_000, 840_000, N)
NW = 32
EPSILON = 1e-7

_mesh = plsc.VectorSubcoreMesh(core_axis_name="c", subcore_axis_name="s")


def _make_part(nanch, lab_off):
    nchunks = nanch // CA

    @functools.partial(
        pl.kernel,
        out_type=jax.ShapeDtypeStruct((NW, 128), jnp.float32),
        mesh=_mesh,
        compiler_params=pltpu.CompilerParams(use_tc_tiling_on_sc=False),
        scratch_types=[
            pltpu.VMEM((2, 4, CA), jnp.float32),
            pltpu.VMEM((2, 4, CA), jnp.float32),
            pltpu.VMEM((2, CA), jnp.int32),
            pltpu.VMEM((128,), jnp.float32),
            pltpu.SemaphoreType.DMA((2,)),
            pltpu.SemaphoreType.DMA((2,)),
            pltpu.SemaphoreType.DMA((2,)),
        ],
    )
    def _part(t_hbm, r_hbm, lab_hbm, out_hbm,
              tv, rv, lv, accv, tsem, rsem, lsem):
        wid = lax.axis_index("s") * 2 + lax.axis_index("c")
        zero = jnp.zeros((16,), jnp.float32)
        one = jnp.ones((16,), jnp.float32)

        nch = jnp.where(wid < (nchunks % NW), nchunks // NW + 1,
                        nchunks // NW)

        def copies(k):
            c = wid + k * NW
            buf = lax.rem(k, 2)
            return (
                pltpu.make_async_copy(t_hbm.at[:, pl.ds(c * CA, CA)],
                                      tv.at[buf], tsem.at[buf]),
                pltpu.make_async_copy(r_hbm.at[:, pl.ds(c * CA, CA)],
                                      rv.at[buf], rsem.at[buf]),
                pltpu.make_async_copy(lab_hbm.at[pl.ds(lab_off + c * CA, CA)],
                                      lv.at[buf], lsem.at[buf]),
            )

        def start(k):
            for cp in copies(k):
                cp.start()

        start(0)

        def chunk_body(k, carry):
            @pl.when(k + 1 < nch)
            def _prefetch():
                start(k + 1)

            for cp in copies(k):
                cp.wait()
            buf = lax.rem(k, 2)

            def group_body(g2, acc):
                acc_a, acc_p, acc_s = acc
                for half in range(2):
                    g = g2 * 2 + half
                    lab16 = lv[buf, pl.ds(g * 16, 16)]
                    w = jnp.where(lab16 == 1, one, zero)
                    acc_p = acc_p + w
                    acc_s = acc_s + lab16.astype(jnp.float32)
                    for j in range(4):
                        t = tv[buf, j, pl.ds(g * 16, 16)]
                        r = rv[buf, j, pl.ds(g * 16, 16)]
                        x = t - r
                        ax = jnp.abs(x)
                        u = jnp.minimum(ax, 1.0)
                        acc_a = acc_a + (u * (ax + ax - u)) * w
                return acc_a, acc_p, acc_s

            return lax.fori_loop(0, CA // 32, group_body, carry)

        acc_a, acc_p, acc_s = lax.fori_loop(0, nch, chunk_body,
                                            (zero, zero, zero))
        accv[pl.ds(0, 16)] = acc_a
        accv[pl.ds(16, 16)] = acc_p
        accv[pl.ds(32, 16)] = acc_s
        accv[pl.ds(48, 16)] = zero
        accv[pl.ds(64, 16)] = zero
        accv[pl.ds(80, 16)] = zero
        accv[pl.ds(96, 16)] = zero
        accv[pl.ds(112, 16)] = zero
        pltpu.sync_copy(accv, out_hbm.at[wid])

    return _part


_NPARTS = len(_BOUNDS) - 1
_parts = [_make_part(_BOUNDS[i + 1] - _BOUNDS[i], _BOUNDS[i])
          for i in range(_NPARTS)]


def kernel(rpn_bbox_targets, rpn_regression, rpn_labels):
    parts = None
    for i in range(_NPARTS):
        lo, hi = _BOUNDS[i], _BOUNDS[i + 1]
        tq = jnp.transpose(lax.slice(rpn_bbox_targets, (lo, 0), (hi, 4)))
        rq = jnp.transpose(lax.slice(rpn_regression, (lo, 0), (hi, 4)))
        p = _parts[i](tq, rq, rpn_labels)
        parts = p if parts is None else parts + p
    a = 0.5 * jnp.sum(parts[:, 0:16])
    npos = jnp.sum(parts[:, 16:32])
    lsum = jnp.sum(parts[:, 32:48])
    nvalid = jnp.float32(N) - npos + lsum
    b = nvalid * EPSILON + npos
    return a / b


# speedup vs baseline: 1.1084x; 1.1084x over previous
import functools

import jax
import jax.numpy as jnp
from jax import lax
from jax.experimental import pallas as pl
from jax.experimental.pallas import tpu as pltpu
from jax.experimental.pallas import tpu_sc as plsc

N = 1_000_000
CA = 4000
_BOUNDS = (0, 248_000, 496_000, 744_000, N)
NW = 32
EPSILON = 1e-7

_mesh = plsc.VectorSubcoreMesh(core_axis_name="c", subcore_axis_name="s")


def _make_part(nanch, lab_off):
    nchunks = nanch // CA

    @functools.partial(
        pl.kernel,
        out_type=jax.ShapeDtypeStruct((NW, 128), jnp.float32),
        mesh=_mesh,
        compiler_params=pltpu.CompilerParams(use_tc_tiling_on_sc=False),
        scratch_types=[
            pltpu.VMEM((2, 4, CA), jnp.float32),
            pltpu.VMEM((2, 4, CA), jnp.float32),
            pltpu.VMEM((2, CA), jnp.int32),
            pltpu.VMEM((128,), jnp.float32),
            pltpu.SemaphoreType.DMA((2,)),
            pltpu.SemaphoreType.DMA((2,)),
            pltpu.SemaphoreType.DMA((2,)),
        ],
    )
    def _part(t_hbm, r_hbm, lab_hbm, out_hbm,
              tv, rv, lv, accv, tsem, rsem, lsem):
        wid = lax.axis_index("s") * 2 + lax.axis_index("c")
        zero = jnp.zeros((16,), jnp.float32)
        one = jnp.ones((16,), jnp.float32)

        nch = jnp.where(wid < (nchunks % NW), nchunks // NW + 1,
                        nchunks // NW)

        def copies(k):
            c = wid + k * NW
            buf = lax.rem(k, 2)
            return (
                pltpu.make_async_copy(t_hbm.at[:, pl.ds(c * CA, CA)],
                                      tv.at[buf], tsem.at[buf]),
                pltpu.make_async_copy(r_hbm.at[:, pl.ds(c * CA, CA)],
                                      rv.at[buf], rsem.at[buf]),
                pltpu.make_async_copy(lab_hbm.at[pl.ds(lab_off + c * CA, CA)],
                                      lv.at[buf], lsem.at[buf]),
            )

        def start(k):
            for cp in copies(k):
                cp.start()

        start(0)

        def chunk_body(k, carry):
            @pl.when(k + 1 < nch)
            def _prefetch():
                start(k + 1)

            for cp in copies(k):
                cp.wait()
            buf = lax.rem(k, 2)

            def group_body(g2, acc):
                acc_a, acc_p, acc_s = acc
                for half in range(2):
                    g = g2 * 2 + half
                    lab16 = lv[buf, pl.ds(g * 16, 16)]
                    w = jnp.where(lab16 == 1, one, zero)
                    acc_p = acc_p + w
                    acc_s = acc_s + lab16.astype(jnp.float32)
                    for j in range(4):
                        t = tv[buf, j, pl.ds(g * 16, 16)]
                        r = rv[buf, j, pl.ds(g * 16, 16)]
                        x = t - r
                        ax = jnp.abs(x)
                        u = jnp.minimum(ax, 1.0)
                        acc_a = acc_a + (u * (ax + ax - u)) * w
                return acc_a, acc_p, acc_s

            return lax.fori_loop(0, CA // 32, group_body, carry)

        acc_a, acc_p, acc_s = lax.fori_loop(0, nch, chunk_body,
                                            (zero, zero, zero))
        accv[pl.ds(0, 16)] = acc_a
        accv[pl.ds(16, 16)] = acc_p
        accv[pl.ds(32, 16)] = acc_s
        accv[pl.ds(48, 16)] = zero
        accv[pl.ds(64, 16)] = zero
        accv[pl.ds(80, 16)] = zero
        accv[pl.ds(96, 16)] = zero
        accv[pl.ds(112, 16)] = zero
        pltpu.sync_copy(accv, out_hbm.at[wid])

    return _part


_NPARTS = len(_BOUNDS) - 1
_parts = [_make_part(_BOUNDS[i + 1] - _BOUNDS[i], _BOUNDS[i])
          for i in range(_NPARTS)]


def kernel(rpn_bbox_targets, rpn_regression, rpn_labels):
    parts = None
    for i in range(_NPARTS):
        lo, hi = _BOUNDS[i], _BOUNDS[i + 1]
        tq = jnp.transpose(lax.slice(rpn_bbox_targets, (lo, 0), (hi, 4)))
        rq = jnp.transpose(lax.slice(rpn_regression, (lo, 0), (hi, 4)))
        p = _parts[i](tq, rq, rpn_labels)
        parts = p if parts is None else parts + p
    a = 0.5 * jnp.sum(parts[:, 0:16])
    npos = jnp.sum(parts[:, 16:32])
    lsum = jnp.sum(parts[:, 32:48])
    nvalid = jnp.float32(N) - npos + lsum
    b = nvalid * EPSILON + npos
    return a / b
